# Initial kernel scaffold; baseline (speedup 1.0000x reference)
import jax
import jax.numpy as jnp
from jax.experimental import pallas as pl


def kernel(energies, species, graph_i, n_graphs, scale, shift):
    raise NotImplementedError("write your pallas kernel here")


# speedup vs baseline: 195.5952x; 195.5952x over previous
import functools

import jax
import jax.numpy as jnp
from jax import lax
from jax.experimental import pallas as pl
from jax.experimental.pallas import tpu as pltpu
from jax.experimental.pallas import tpu_sc as plsc

N = 6_400_000
NUM_ELEMENTS = 89
TBL = 96
N_GRAPHS = 4096
NC = 2
NS = 16
NW = NC * NS
PER_W = N // NW
CHUNK = 4000
N_CHUNKS = PER_W // CHUNK
VECS = CHUNK // 16
GPW = N_GRAPHS // NW

_mesh = plsc.VectorSubcoreMesh(core_axis_name="c", subcore_axis_name="s")
_cparams = pltpu.CompilerParams(needs_layout_passes=False)


def _seg_body(en_hbm, sp_hbm, g_hbm, scale_hbm, shift_hbm,
              psums_hbm, pcnts_hbm,
              en_buf, sp_buf, g_buf, scale_v, shift_v, sums_acc, cnts_acc):
    wid = lax.axis_index("s") * NC + lax.axis_index("c")
    base = wid * PER_W

    pltpu.sync_copy(scale_hbm, scale_v)
    pltpu.sync_copy(shift_hbm, shift_v)

    zeros = jnp.zeros((16,), jnp.float32)

    def zbody(i, carry):
        sums_acc[pl.ds(i * 16, 16)] = zeros
        cnts_acc[pl.ds(i * 16, 16)] = zeros
        return carry

    lax.fori_loop(0, N_GRAPHS // 16, zbody, 0)

    ones = jnp.ones((16,), jnp.float32)

    def chunk_body(ci, carry):
        off = base + ci * CHUNK
        pltpu.sync_copy(en_hbm.at[pl.ds(off, CHUNK)], en_buf)
        pltpu.sync_copy(sp_hbm.at[pl.ds(off, CHUNK)], sp_buf)
        pltpu.sync_copy(g_hbm.at[pl.ds(off, CHUNK)], g_buf)

        def vbody(v, c2):
            sl = pl.ds(v * 16, 16)
            en = en_buf[sl]
            sp = sp_buf[sl]
            g = g_buf[sl]
            sc = plsc.load_gather(scale_v, [sp])
            sh = plsc.load_gather(shift_v, [sp])
            e = en * sc + sh
            plsc.addupdate_scatter(sums_acc, [g], e)
            plsc.addupdate_scatter(cnts_acc, [g], ones)
            return c2

        lax.fori_loop(0, VECS, vbody, 0)
        return carry

    lax.fori_loop(0, N_CHUNKS, chunk_body, 0)

    pltpu.sync_copy(sums_acc, psums_hbm.at[wid])
    pltpu.sync_copy(cnts_acc, pcnts_hbm.at[wid])


def _comb_body(psums_hbm, pcnts_hbm, out_hbm, sbuf, cbuf, obuf):
    wid = lax.axis_index("s") * NC + lax.axis_index("c")
    g0 = wid * GPW

    pltpu.sync_copy(psums_hbm.at[:, pl.ds(g0, GPW)], sbuf)
    pltpu.sync_copy(pcnts_hbm.at[:, pl.ds(g0, GPW)], cbuf)

    def vbody(v, carry):
        sl = pl.ds(v * 16, 16)

        def rbody(r, acc):
            return (acc[0] + sbuf[r, sl], acc[1] + cbuf[r, sl])

        ssum, csum = lax.fori_loop(
            0, NW, rbody,
            (jnp.zeros((16,), jnp.float32), jnp.zeros((16,), jnp.float32)))
        obuf[sl] = ssum / jnp.maximum(csum, 1.0)
        return carry

    lax.fori_loop(0, GPW // 16, vbody, 0)
    pltpu.sync_copy(obuf, out_hbm.at[pl.ds(g0, GPW)])


_seg = pl.kernel(
    _seg_body,
    mesh=_mesh,
    compiler_params=_cparams,
    out_type=(
        jax.ShapeDtypeStruct((NW, N_GRAPHS), jnp.float32),
        jax.ShapeDtypeStruct((NW, N_GRAPHS), jnp.float32),
    ),
    scratch_types=[
        pltpu.VMEM((CHUNK,), jnp.float32),
        pltpu.VMEM((CHUNK,), jnp.int32),
        pltpu.VMEM((CHUNK,), jnp.int32),
        pltpu.VMEM((TBL,), jnp.float32),
        pltpu.VMEM((TBL,), jnp.float32),
        pltpu.VMEM((N_GRAPHS,), jnp.float32),
        pltpu.VMEM((N_GRAPHS,), jnp.float32),
    ],
)

_comb = pl.kernel(
    _comb_body,
    mesh=_mesh,
    compiler_params=_cparams,
    out_type=jax.ShapeDtypeStruct((N_GRAPHS,), jnp.float32),
    scratch_types=[
        pltpu.VMEM((NW, GPW), jnp.float32),
        pltpu.VMEM((NW, GPW), jnp.float32),
        pltpu.VMEM((GPW,), jnp.float32),
    ],
)


def kernel(energies, species, graph_i, n_graphs, scale, shift):
    del n_graphs
    pad = jnp.zeros((TBL - NUM_ELEMENTS,), jnp.float32)
    scale_p = jnp.concatenate([scale.astype(jnp.float32), pad])
    shift_p = jnp.concatenate([shift.astype(jnp.float32), pad])
    psums, pcnts = _seg(
        energies.astype(jnp.float32),
        species.astype(jnp.int32),
        graph_i.astype(jnp.int32),
        scale_p, shift_p)
    out = _comb(psums, pcnts)
    return out[:, None]


# speedup vs baseline: 270.5439x; 1.3832x over previous
import functools

import jax
import jax.numpy as jnp
from jax import lax
from jax.experimental import pallas as pl
from jax.experimental.pallas import tpu as pltpu
from jax.experimental.pallas import tpu_sc as plsc

N = 6_400_000
NUM_ELEMENTS = 89
TBL = 96
N_GRAPHS = 4096
NC = 2
NS = 16
NW = NC * NS
PER_W = N // NW
CHUNK = 4000
N_CHUNKS = PER_W // CHUNK
VECS = CHUNK // 16
GPW = N_GRAPHS // NW

_mesh = plsc.VectorSubcoreMesh(core_axis_name="c", subcore_axis_name="s")
_cparams = pltpu.CompilerParams(needs_layout_passes=False)


UNROLL = 10


def _seg_body(en_hbm, sp_hbm, g_hbm, scale_hbm, shift_hbm,
              psums_hbm, pcnts_hbm,
              en0, sp0, g0, en1, sp1, g1,
              scale_v, shift_v, sums_acc, cnts_acc, sem0, sem1):
    wid = lax.axis_index("s") * NC + lax.axis_index("c")
    base = wid * PER_W

    pltpu.sync_copy(scale_hbm, scale_v)
    pltpu.sync_copy(shift_hbm, shift_v)

    def start(ci, en_b, sp_b, g_b, sem):
        off = base + ci * CHUNK
        pltpu.make_async_copy(en_hbm.at[pl.ds(off, CHUNK)], en_b, sem).start()
        pltpu.make_async_copy(sp_hbm.at[pl.ds(off, CHUNK)], sp_b, sem).start()
        pltpu.make_async_copy(g_hbm.at[pl.ds(off, CHUNK)], g_b, sem).start()

    def wait(en_b, sp_b, g_b, sem):
        pltpu.make_async_copy(en_hbm.at[pl.ds(0, CHUNK)], en_b, sem).wait()
        pltpu.make_async_copy(sp_hbm.at[pl.ds(0, CHUNK)], sp_b, sem).wait()
        pltpu.make_async_copy(g_hbm.at[pl.ds(0, CHUNK)], g_b, sem).wait()

    start(0, en0, sp0, g0, sem0)
    start(1, en1, sp1, g1, sem1)

    zeros = jnp.zeros((16,), jnp.float32)

    def zbody(i, carry):
        sums_acc[pl.ds(i * 16, 16)] = zeros
        cnts_acc[pl.ds(i * 16, 16)] = zeros
        return carry

    lax.fori_loop(0, N_GRAPHS // 16, zbody, 0)

    ones = jnp.ones((16,), jnp.float32)

    def compute(en_b, sp_b, g_b):
        @plsc.parallel_loop(0, VECS, unroll=UNROLL)
        def _(v):
            sl = pl.ds(v * 16, 16)
            en = en_b[sl]
            sp = sp_b[sl]
            g = g_b[sl]
            sc = plsc.load_gather(scale_v, [sp])
            sh = plsc.load_gather(shift_v, [sp])
            e = en * sc + sh
            plsc.addupdate_scatter(sums_acc, [g], e)
            plsc.addupdate_scatter(cnts_acc, [g], ones)

    def pair_body(p, carry):
        wait(en0, sp0, g0, sem0)
        compute(en0, sp0, g0)

        @pl.when(2 * p + 2 < N_CHUNKS)
        def _():
            start(2 * p + 2, en0, sp0, g0, sem0)

        wait(en1, sp1, g1, sem1)
        compute(en1, sp1, g1)

        @pl.when(2 * p + 3 < N_CHUNKS)
        def _():
            start(2 * p + 3, en1, sp1, g1, sem1)

        return carry

    lax.fori_loop(0, N_CHUNKS // 2, pair_body, 0)

    pltpu.sync_copy(sums_acc, psums_hbm.at[wid])
    pltpu.sync_copy(cnts_acc, pcnts_hbm.at[wid])


def _comb_body(psums_hbm, pcnts_hbm, out_hbm, sbuf, cbuf, obuf):
    wid = lax.axis_index("s") * NC + lax.axis_index("c")
    g0 = wid * GPW

    pltpu.sync_copy(psums_hbm.at[:, pl.ds(g0, GPW)], sbuf)
    pltpu.sync_copy(pcnts_hbm.at[:, pl.ds(g0, GPW)], cbuf)

    def vbody(v, carry):
        sl = pl.ds(v * 16, 16)

        def rbody(r, acc):
            return (acc[0] + sbuf[r, sl], acc[1] + cbuf[r, sl])

        ssum, csum = lax.fori_loop(
            0, NW, rbody,
            (jnp.zeros((16,), jnp.float32), jnp.zeros((16,), jnp.float32)))
        obuf[sl] = ssum / jnp.maximum(csum, 1.0)
        return carry

    lax.fori_loop(0, GPW // 16, vbody, 0)
    pltpu.sync_copy(obuf, out_hbm.at[pl.ds(g0, GPW)])


_seg = pl.kernel(
    _seg_body,
    mesh=_mesh,
    compiler_params=_cparams,
    out_type=(
        jax.ShapeDtypeStruct((NW, N_GRAPHS), jnp.float32),
        jax.ShapeDtypeStruct((NW, N_GRAPHS), jnp.float32),
    ),
    scratch_types=[
        pltpu.VMEM((CHUNK,), jnp.float32),
        pltpu.VMEM((CHUNK,), jnp.int32),
        pltpu.VMEM((CHUNK,), jnp.int32),
        pltpu.VMEM((CHUNK,), jnp.float32),
        pltpu.VMEM((CHUNK,), jnp.int32),
        pltpu.VMEM((CHUNK,), jnp.int32),
        pltpu.VMEM((TBL,), jnp.float32),
        pltpu.VMEM((TBL,), jnp.float32),
        pltpu.VMEM((N_GRAPHS,), jnp.float32),
        pltpu.VMEM((N_GRAPHS,), jnp.float32),
        pltpu.SemaphoreType.DMA,
        pltpu.SemaphoreType.DMA,
    ],
)

_comb = pl.kernel(
    _comb_body,
    mesh=_mesh,
    compiler_params=_cparams,
    out_type=jax.ShapeDtypeStruct((N_GRAPHS,), jnp.float32),
    scratch_types=[
        pltpu.VMEM((NW, GPW), jnp.float32),
        pltpu.VMEM((NW, GPW), jnp.float32),
        pltpu.VMEM((GPW,), jnp.float32),
    ],
)


def kernel(energies, species, graph_i, n_graphs, scale, shift):
    del n_graphs
    pad = jnp.zeros((TBL - NUM_ELEMENTS,), jnp.float32)
    scale_p = jnp.concatenate([scale.astype(jnp.float32), pad])
    shift_p = jnp.concatenate([shift.astype(jnp.float32), pad])
    psums, pcnts = _seg(
        energies.astype(jnp.float32),
        species.astype(jnp.int32),
        graph_i.astype(jnp.int32),
        scale_p, shift_p)
    out = _comb(psums, pcnts)
    return out[:, None]
